# Initial kernel scaffold; baseline (speedup 1.0000x reference)
import jax
import jax.numpy as jnp
from jax.experimental import pallas as pl


def kernel(x, edge_index, W1, b1, g1, be1, W1s, b1s, g1s, be1s, W2, b2, g2, be2, W2s, b2s, g2s, be2s, W3, b3, g3, be3, W3s, b3s, g3s, be3s):
    raise NotImplementedError("write your pallas kernel here")


# speedup vs baseline: 36.3405x; 36.3405x over previous
import functools

import jax
import jax.numpy as jnp
from jax import lax
from jax.experimental import pallas as pl
from jax.experimental.pallas import tpu as pltpu
from jax.experimental.pallas import tpu_sc as plsc

L = 1024
E = 8192
NC = 2
NS = 16
EPW = E // (NC * NS)
WPS = (L * L) // NS
ZCH = 8192


def _sc_counts(src, dst):
    mesh = plsc.VectorSubcoreMesh(core_axis_name="c", subcore_axis_name="s")

    @functools.partial(
        pl.kernel,
        mesh=mesh,
        out_type=jax.ShapeDtypeStruct((NC, L * L), jnp.float32),
        scratch_types=[
            pltpu.VMEM((EPW,), jnp.int32),
            pltpu.VMEM((EPW,), jnp.int32),
            pltpu.VMEM((EPW // 128, 128), jnp.int32),
            pltpu.VMEM((128,), jnp.float32),
            pltpu.VMEM((ZCH,), jnp.float32),
            pltpu.VMEM_SHARED((L * L,), jnp.float32),
        ],
    )
    def k(src_hbm, dst_hbm, out_hbm, sv, dv, iv, ones_v, zv, csh):
        cid = lax.axis_index("c")
        sid = lax.axis_index("s")

        def fill16(i, ref, val):
            ref[pl.ds(i * 16, 16)] = jnp.full((16,), val, ref.dtype)

        lax.fori_loop(0, ZCH // 16, lambda i, c: (fill16(i, zv, 0.0), c)[1], 0)
        lax.fori_loop(0, 128 // 16, lambda i, c: (fill16(i, ones_v, 1.0), c)[1], 0)

        base = sid * WPS

        def zc(j, c):
            pltpu.sync_copy(zv, csh.at[pl.ds(base + j * ZCH, ZCH)])
            return c

        lax.fori_loop(0, WPS // ZCH, zc, 0)
        plsc.subcore_barrier()

        ebase = (cid * NS + sid) * EPW
        pltpu.sync_copy(src_hbm.at[pl.ds(ebase, EPW)], sv)
        pltpu.sync_copy(dst_hbm.at[pl.ds(ebase, EPW)], dv)
        for g in range(EPW // 128):
            for j in range(128 // 16):
                s16 = sv[pl.ds(g * 128 + j * 16, 16)]
                d16 = dv[pl.ds(g * 128 + j * 16, 16)]
                iv[g, pl.ds(j * 16, 16)] = s16 * L + d16

        for g in range(EPW // 128):
            pltpu.sync_copy(ones_v, csh.at[iv.at[g]], add=True)
        plsc.subcore_barrier()

        def co(j, c):
            pltpu.sync_copy(csh.at[pl.ds(base + j * ZCH, ZCH)],
                            out_hbm.at[cid, pl.ds(base + j * ZCH, ZCH)])
            return c

        lax.fori_loop(0, WPS // ZCH, co, 0)

    return k(src, dst)


def _prep_body(ct2_ref, ct_ref, dinv_ref, dinv2_ref):
    ct = ct2_ref[0] + ct2_ref[1]
    ct_ref[...] = ct
    deg = jnp.sum(ct, axis=0, keepdims=True) + 2.0
    di = lax.rsqrt(deg)
    dinv_ref[...] = di
    dinv2_ref[...] = 2.0 * di * di


def _prep(counts2):
    return pl.pallas_call(
        _prep_body,
        out_shape=[
            jax.ShapeDtypeStruct((L, L), jnp.float32),
            jax.ShapeDtypeStruct((1, L), jnp.float32),
            jax.ShapeDtypeStruct((1, L), jnp.float32),
        ],
    )(counts2)


def _stage_body(G, B, x_ref, ct_ref, dinv_ref, dinv2_ref,
                wt_ref, b_ref, g_ref, be_ref,
                wst_ref, bs_ref, gs_ref, bes_ref,
                out_ref, zs_ref):
    cout = out_ref.shape[1]
    n = G // B
    di = dinv_ref[...]
    di2 = dinv2_ref[...]
    ct = ct_ref[...]
    wt = wt_ref[...]
    wst = wst_ref[...]
    b = b_ref[...]
    bs = bs_ref[...]

    def conv(m):
        t = jnp.dot(m * di, ct, preferred_element_type=jnp.float32)
        return t * di + m * di2

    def body1(g, carry):
        ssum, ssq = carry
        z = jnp.dot(wt, conv(x_ref[g]),
                    preferred_element_type=jnp.float32) + b
        out_ref[g] = z
        return (ssum + jnp.sum(z, axis=1, keepdims=True),
                ssq + jnp.sum(z * z, axis=1, keepdims=True))

    zc = jnp.zeros((cout, 1), jnp.float32)
    ssum, ssq = lax.fori_loop(0, G, body1, (zc, zc))
    mean = ssum / (G * L)
    var = ssq / (G * L) - mean * mean
    rstd = lax.rsqrt(var + 1e-5)

    def body2(bb, carry):
        ssum, ssq = carry
        xs = x_ref[n * bb]
        for i in range(1, n):
            xs = xs + x_ref[n * bb + i]
        z2 = jnp.dot(wst, conv(xs),
                     preferred_element_type=jnp.float32) + bs
        zs_ref[bb] = z2
        return (ssum + jnp.sum(z2, axis=1, keepdims=True),
                ssq + jnp.sum(z2 * z2, axis=1, keepdims=True))

    s2sum, s2sq = lax.fori_loop(0, B, body2, (zc, zc))
    mean2 = s2sum / (B * L)
    var2 = s2sq / (B * L) - mean2 * mean2
    rstd2 = lax.rsqrt(var2 + 1e-5)

    sc1 = rstd * g_ref[...]
    of1 = be_ref[...] - mean * sc1
    sc2 = rstd2 * gs_ref[...]
    of2 = bes_ref[...] - mean2 * sc2

    def body3(g, c):
        z1 = out_ref[g] * sc1 + of1
        z2 = zs_ref[g // n] * sc2 + of2
        out_ref[g] = jnp.maximum(z1 + z2, 0.0)
        return c

    lax.fori_loop(0, G, body3, 0)


def _stage(x, ct, dinv, dinv2, wt, b, gam, bet, wst, bs, gs, bes):
    cout = wt.shape[0]
    G = x.shape[0]
    B = 4
    return pl.pallas_call(
        functools.partial(_stage_body, G, B),
        out_shape=jax.ShapeDtypeStruct((G, cout, L), jnp.float32),
        scratch_shapes=[pltpu.VMEM((B, cout, L), jnp.float32)],
    )(x, ct, dinv, dinv2, wt, b, gam, bet, wst, bs, gs, bes)


def _col(v):
    return v.reshape(-1, 1)


def kernel(x, edge_index, W1, b1, g1, be1, W1s, b1s, g1s, be1s,
           W2, b2, g2, be2, W2s, b2s, g2s, be2s,
           W3, b3, g3, be3, W3s, b3s, g3s, be3s):
    ei = edge_index.astype(jnp.int32)
    counts2 = _sc_counts(ei[0], ei[1])
    ct, dinv, dinv2 = _prep(counts2.reshape(NC, L, L))

    h = x.reshape(16, x.shape[2], L)
    h = _stage(h, ct, dinv, dinv2, W1.T, _col(b1), _col(g1), _col(be1),
               W1s.T, _col(b1s), _col(g1s), _col(be1s))
    h = _stage(h, ct, dinv, dinv2, W2.T, _col(b2), _col(g2), _col(be2),
               W2s.T, _col(b2s), _col(g2s), _col(be2s))
    h = _stage(h, ct, dinv, dinv2, W3.T, _col(b3), _col(g3), _col(be3),
               W3s.T, _col(b3s), _col(g3s), _col(be3s))
    return h


# speedup vs baseline: 36.6557x; 1.0087x over previous
import functools

import jax
import jax.numpy as jnp
from jax import lax
from jax.experimental import pallas as pl
from jax.experimental.pallas import tpu as pltpu
from jax.experimental.pallas import tpu_sc as plsc

L = 1024
E = 8192
NC = 2
NS = 16
EPW = E // (NC * NS)
WPS = (L * L) // NS
ZCH = 8192


def _sc_counts(src, dst):
    mesh = plsc.VectorSubcoreMesh(core_axis_name="c", subcore_axis_name="s")

    @functools.partial(
        pl.kernel,
        mesh=mesh,
        out_type=jax.ShapeDtypeStruct((NC, L * L), jnp.float32),
        scratch_types=[
            pltpu.VMEM((EPW,), jnp.int32),
            pltpu.VMEM((EPW,), jnp.int32),
            pltpu.VMEM((EPW // 128, 128), jnp.int32),
            pltpu.VMEM((128,), jnp.float32),
            pltpu.VMEM((ZCH,), jnp.float32),
            pltpu.VMEM_SHARED((L * L,), jnp.float32),
        ],
    )
    def k(src_hbm, dst_hbm, out_hbm, sv, dv, iv, ones_v, zv, csh):
        cid = lax.axis_index("c")
        sid = lax.axis_index("s")

        def fill16(i, ref, val):
            ref[pl.ds(i * 16, 16)] = jnp.full((16,), val, ref.dtype)

        lax.fori_loop(0, ZCH // 16, lambda i, c: (fill16(i, zv, 0.0), c)[1], 0)
        lax.fori_loop(0, 128 // 16, lambda i, c: (fill16(i, ones_v, 1.0), c)[1], 0)

        base = sid * WPS

        def zc(j, c):
            pltpu.sync_copy(zv, csh.at[pl.ds(base + j * ZCH, ZCH)])
            return c

        lax.fori_loop(0, WPS // ZCH, zc, 0)
        plsc.subcore_barrier()

        ebase = (cid * NS + sid) * EPW
        pltpu.sync_copy(src_hbm.at[pl.ds(ebase, EPW)], sv)
        pltpu.sync_copy(dst_hbm.at[pl.ds(ebase, EPW)], dv)
        for g in range(EPW // 128):
            for j in range(128 // 16):
                s16 = sv[pl.ds(g * 128 + j * 16, 16)]
                d16 = dv[pl.ds(g * 128 + j * 16, 16)]
                iv[g, pl.ds(j * 16, 16)] = s16 * L + d16

        for g in range(EPW // 128):
            pltpu.sync_copy(ones_v, csh.at[iv.at[g]], add=True)
        plsc.subcore_barrier()

        def co(j, c):
            pltpu.sync_copy(csh.at[pl.ds(base + j * ZCH, ZCH)],
                            out_hbm.at[cid, pl.ds(base + j * ZCH, ZCH)])
            return c

        lax.fori_loop(0, WPS // ZCH, co, 0)

    return k(src, dst)


def _prep_body(ct2_ref, ct_ref, dinv_ref, dinv2_ref):
    ct = ct2_ref[0] + ct2_ref[1]
    ct_ref[...] = ct
    deg = jnp.sum(ct, axis=0, keepdims=True) + 2.0
    di = lax.rsqrt(deg)
    dinv_ref[...] = di
    dinv2_ref[...] = 2.0 * di * di


def _prep(counts2):
    return pl.pallas_call(
        _prep_body,
        out_shape=[
            jax.ShapeDtypeStruct((L, L), jnp.float32),
            jax.ShapeDtypeStruct((1, L), jnp.float32),
            jax.ShapeDtypeStruct((1, L), jnp.float32),
        ],
    )(counts2)


def _stage_body(G, B, x_ref, ct_ref, dinv_ref, dinv2_ref,
                wt_ref, b_ref, g_ref, be_ref,
                wst_ref, bs_ref, gs_ref, bes_ref,
                out_ref, zs_ref, ts_ref):
    cout = out_ref.shape[1]
    n = G // B
    di = dinv_ref[...]
    di2 = dinv2_ref[...]
    ct = ct_ref[...].astype(jnp.bfloat16)
    wt = wt_ref[...].astype(jnp.bfloat16)
    wst = wst_ref[...].astype(jnp.bfloat16)
    b = b_ref[...]
    bs = bs_ref[...]

    def conv(m):
        t = jnp.dot((m * di).astype(jnp.bfloat16), ct,
                    preferred_element_type=jnp.float32)
        return t * di + m * di2

    def body1(g, carry):
        ssum, ssq = carry
        t = conv(x_ref[g])
        bb = g // n

        @pl.when(g % n == 0)
        def _():
            ts_ref[bb] = t

        @pl.when(g % n != 0)
        def _():
            ts_ref[bb] = ts_ref[bb] + t

        z = jnp.dot(wt, t.astype(jnp.bfloat16),
                    preferred_element_type=jnp.float32) + b
        out_ref[g] = z
        return (ssum + jnp.sum(z, axis=1, keepdims=True),
                ssq + jnp.sum(z * z, axis=1, keepdims=True))

    zc = jnp.zeros((cout, 1), jnp.float32)
    ssum, ssq = lax.fori_loop(0, G, body1, (zc, zc))
    mean = ssum / (G * L)
    var = ssq / (G * L) - mean * mean
    rstd = lax.rsqrt(var + 1e-5)

    def body2(bb, carry):
        ssum, ssq = carry
        z2 = jnp.dot(wst, ts_ref[bb].astype(jnp.bfloat16),
                     preferred_element_type=jnp.float32) + bs
        zs_ref[bb] = z2
        return (ssum + jnp.sum(z2, axis=1, keepdims=True),
                ssq + jnp.sum(z2 * z2, axis=1, keepdims=True))

    s2sum, s2sq = lax.fori_loop(0, B, body2, (zc, zc))
    mean2 = s2sum / (B * L)
    var2 = s2sq / (B * L) - mean2 * mean2
    rstd2 = lax.rsqrt(var2 + 1e-5)

    sc1 = rstd * g_ref[...]
    of1 = be_ref[...] - mean * sc1
    sc2 = rstd2 * gs_ref[...]
    of2 = bes_ref[...] - mean2 * sc2

    def body3(g, c):
        z1 = out_ref[g] * sc1 + of1
        z2 = zs_ref[g // n] * sc2 + of2
        out_ref[g] = jnp.maximum(z1 + z2, 0.0)
        return c

    lax.fori_loop(0, G, body3, 0)


def _stage(x, ct, dinv, dinv2, wt, b, gam, bet, wst, bs, gs, bes):
    cout = wt.shape[0]
    cin = wt.shape[1]
    G = x.shape[0]
    B = 4
    return pl.pallas_call(
        functools.partial(_stage_body, G, B),
        out_shape=jax.ShapeDtypeStruct((G, cout, L), jnp.float32),
        scratch_shapes=[pltpu.VMEM((B, cout, L), jnp.float32),
                        pltpu.VMEM((B, cin, L), jnp.float32)],
    )(x, ct, dinv, dinv2, wt, b, gam, bet, wst, bs, gs, bes)


def _col(v):
    return v.reshape(-1, 1)


def kernel(x, edge_index, W1, b1, g1, be1, W1s, b1s, g1s, be1s,
           W2, b2, g2, be2, W2s, b2s, g2s, be2s,
           W3, b3, g3, be3, W3s, b3s, g3s, be3s):
    ei = edge_index.astype(jnp.int32)
    counts2 = _sc_counts(ei[0], ei[1])
    ct, dinv, dinv2 = _prep(counts2.reshape(NC, L, L))

    h = x.reshape(16, x.shape[2], L)
    h = _stage(h, ct, dinv, dinv2, W1.T, _col(b1), _col(g1), _col(be1),
               W1s.T, _col(b1s), _col(g1s), _col(be1s))
    h = _stage(h, ct, dinv, dinv2, W2.T, _col(b2), _col(g2), _col(be2),
               W2s.T, _col(b2s), _col(g2s), _col(be2s))
    h = _stage(h, ct, dinv, dinv2, W3.T, _col(b3), _col(g3), _col(be3),
               W3s.T, _col(b3s), _col(g3s), _col(be3s))
    return h
